# baseline (speedup 1.0000x reference)
import functools

import jax
import jax.numpy as jnp
from jax import lax
from jax.experimental import pallas as pl
from jax.experimental.pallas import tpu as pltpu
from jax.experimental.pallas import tpu_sc as plsc

B = 16384
K = 128

NC = 2
NS = 16
NW = NC * NS
COLS_PER_W = B // NW
CHUNK = 128
NCH = COLS_PER_W // CHUNK


def _sc_gather_t(flat_table, idx, n_emb):
    mesh = plsc.VectorSubcoreMesh(core_axis_name="c", subcore_axis_name="s")
    L = 16

    @functools.partial(
        pl.kernel,
        mesh=mesh,
        compiler_params=pltpu.CompilerParams(use_tc_tiling_on_sc=False),
        out_type=jax.ShapeDtypeStruct((3, B), jnp.float32),
        scratch_types=[
            pltpu.VMEM((3, NCH, CHUNK), jnp.int32),
            pltpu.VMEM((3, NCH, CHUNK), jnp.float32),
            pltpu.SemaphoreType.DMA,
            pltpu.SemaphoreType.DMA,
            pltpu.SemaphoreType.DMA,
        ],
    )
    def gather_kernel(table_hbm, idx_hbm, out_hbm, idx_v, vals_v,
                      sem_i, sem_g, sem_o):
        wid = lax.axis_index("s") * NC + lax.axis_index("c")
        idx_loads = [
            pltpu.async_copy(
                idx_hbm.at[pl.ds(wid * COLS_PER_W + j * CHUNK, CHUNK)],
                idx_v.at[0, j],
                sem_i,
            )
            for j in range(NCH)
        ]
        for cp in idx_loads:
            cp.wait()
        for c in (1, 2):
            for j in range(NCH):
                for k in range(CHUNK // L):
                    idx_v[c, j, pl.ds(k * L, L)] = (
                        idx_v[0, j, pl.ds(k * L, L)] + c * n_emb)
        gathers = [
            pltpu.async_copy(
                table_hbm.at[idx_v.at[c, j]],
                vals_v.at[c, j],
                sem_g,
            )
            for c in range(3)
            for j in range(NCH)
        ]
        writes = []
        for c in range(3):
            for j in range(NCH):
                gathers[c * NCH + j].wait()
                writes.append(pltpu.async_copy(
                    vals_v.at[c, j],
                    out_hbm.at[c, pl.ds(wid * COLS_PER_W + j * CHUNK, CHUNK)],
                    sem_o,
                ))
        for cp in writes:
            cp.wait()

    return gather_kernel(flat_table, idx)


BLK = 8192


def _tc_mlp_t(x, w0, w1, w2, bf):
    def body(x_ref, w0_ref, w1_ref, w2_ref, bf_ref, out_ref):
        wf = lax.dot_general(w1_ref[...], w0_ref[...], (((1,), (0,)), ((), ())),
                             preferred_element_type=jnp.float32)
        wf = lax.dot_general(w2_ref[...], wf, (((1,), (0,)), ((), ())),
                             preferred_element_type=jnp.float32)
        m = lax.dot_general(wf, x_ref[...], (((1,), (1,)), ((), ())),
                            preferred_element_type=jnp.float32)
        out_ref[...] = m + bf_ref[...]

    return pl.pallas_call(
        body,
        grid=(B // BLK,),
        in_specs=[
            pl.BlockSpec((BLK, K), lambda i: (i, 0)),
            pl.BlockSpec((12, K), lambda i: (0, 0)),
            pl.BlockSpec((6, 12), lambda i: (0, 0)),
            pl.BlockSpec((3, 6), lambda i: (0, 0)),
            pl.BlockSpec((3, 1), lambda i: (0, 0)),
        ],
        out_specs=pl.BlockSpec((3, BLK), lambda i: (0, i)),
        out_shape=jax.ShapeDtypeStruct((3, B), jnp.float32),
    )(x, w0, w1, w2, bf)


def kernel(eb_input, eb_offset, mlp_input, eb_weight, W0, b0, W1, b1, W2, b2):
    del eb_offset
    n_emb = eb_weight.shape[0]
    flat_t = eb_weight.T.reshape(3 * n_emb)
    idx = eb_input.astype(jnp.int32)
    bag_t = _sc_gather_t(flat_t, idx, n_emb)
    bf = (W2 @ (W1 @ b0 + b1) + b2).reshape(3, 1)
    mlp_t = _tc_mlp_t(mlp_input, W0, W1, W2, bf)
    return jnp.concatenate([bag_t, bag_t, bag_t, mlp_t], axis=0).T


# speedup vs baseline: 1.0321x; 1.0321x over previous
import functools

import jax
import jax.numpy as jnp
from jax import lax
from jax.experimental import pallas as pl
from jax.experimental.pallas import tpu as pltpu
from jax.experimental.pallas import tpu_sc as plsc

B = 16384
K = 128

NC = 2
NS = 16
NW = NC * NS
COLS_PER_W = B // NW
CHUNK = 128
NCH = COLS_PER_W // CHUNK


def _sc_gather_t(flat_table, idx, n_emb):
    mesh = plsc.VectorSubcoreMesh(core_axis_name="c", subcore_axis_name="s")
    L = 16

    @functools.partial(
        pl.kernel,
        mesh=mesh,
        compiler_params=pltpu.CompilerParams(use_tc_tiling_on_sc=False),
        out_type=jax.ShapeDtypeStruct((9, B), jnp.float32),
        scratch_types=[
            pltpu.VMEM((3, NCH, CHUNK), jnp.int32),
            pltpu.VMEM((3, NCH, CHUNK), jnp.float32),
            pltpu.SemaphoreType.DMA,
            pltpu.SemaphoreType.DMA,
            pltpu.SemaphoreType.DMA,
        ],
    )
    def gather_kernel(table_hbm, idx_hbm, out_hbm, idx_v, vals_v,
                      sem_i, sem_g, sem_o):
        wid = lax.axis_index("s") * NC + lax.axis_index("c")
        idx_loads = [
            pltpu.async_copy(
                idx_hbm.at[pl.ds(wid * COLS_PER_W + j * CHUNK, CHUNK)],
                idx_v.at[0, j],
                sem_i,
            )
            for j in range(NCH)
        ]
        for cp in idx_loads:
            cp.wait()
        for c in (1, 2):
            for j in range(NCH):
                for k in range(CHUNK // L):
                    idx_v[c, j, pl.ds(k * L, L)] = (
                        idx_v[0, j, pl.ds(k * L, L)] + c * n_emb)
        gathers = [
            pltpu.async_copy(
                table_hbm.at[idx_v.at[c, j]],
                vals_v.at[c, j],
                sem_g,
            )
            for c in range(3)
            for j in range(NCH)
        ]
        writes = []
        for c in range(3):
            for j in range(NCH):
                gathers[c * NCH + j].wait()
                for t in range(3):
                    writes.append(pltpu.async_copy(
                        vals_v.at[c, j],
                        out_hbm.at[3 * t + c,
                                   pl.ds(wid * COLS_PER_W + j * CHUNK, CHUNK)],
                        sem_o,
                    ))
        for cp in writes:
            cp.wait()

    return gather_kernel(flat_table, idx)


BLK = 8192


def _tc_mlp_t(x, w0, w1, w2, bf):
    def body(x_ref, w0_ref, w1_ref, w2_ref, bf_ref, out_ref):
        wf = lax.dot_general(w1_ref[...], w0_ref[...], (((1,), (0,)), ((), ())),
                             preferred_element_type=jnp.float32)
        wf = lax.dot_general(w2_ref[...], wf, (((1,), (0,)), ((), ())),
                             preferred_element_type=jnp.float32)
        m = lax.dot_general(wf, x_ref[...], (((1,), (1,)), ((), ())),
                            preferred_element_type=jnp.float32)
        out_ref[...] = m + bf_ref[...]

    return pl.pallas_call(
        body,
        grid=(B // BLK,),
        in_specs=[
            pl.BlockSpec((BLK, K), lambda i: (i, 0)),
            pl.BlockSpec((12, K), lambda i: (0, 0)),
            pl.BlockSpec((6, 12), lambda i: (0, 0)),
            pl.BlockSpec((3, 6), lambda i: (0, 0)),
            pl.BlockSpec((3, 1), lambda i: (0, 0)),
        ],
        out_specs=pl.BlockSpec((3, BLK), lambda i: (0, i)),
        out_shape=jax.ShapeDtypeStruct((3, B), jnp.float32),
    )(x, w0, w1, w2, bf)


def kernel(eb_input, eb_offset, mlp_input, eb_weight, W0, b0, W1, b1, W2, b2):
    del eb_offset
    n_emb = eb_weight.shape[0]
    flat_t = eb_weight.T.reshape(3 * n_emb)
    idx = eb_input.astype(jnp.int32)
    bag_t = _sc_gather_t(flat_t, idx, n_emb)
    bf = (W2 @ (W1 @ b0 + b1) + b2).reshape(3, 1)
    mlp_t = _tc_mlp_t(mlp_input, W0, W1, W2, bf)
    return jnp.concatenate([bag_t, mlp_t], axis=0).T
